# baseline (speedup 1.0000x reference)
import functools

import jax
import jax.numpy as jnp
from jax import lax
from jax.experimental import pallas as pl
from jax.experimental.pallas import tpu as pltpu
from jax.experimental.pallas import tpu_sc as plsc

B = 16384
N_USERS = 1000000
D = 16
F = 26
NC = 2
NS = 16
NW = NC * NS
BPW = B // NW
G = 16
C = 64
NCHUNK = BPW // C
CG = C // G


def _rsqrt16(n2):
    i = plsc.bitcast(n2, jnp.int32)
    y = plsc.bitcast(jnp.int32(0x5F3759DF) - jnp.right_shift(i, 1),
                     jnp.float32)
    for _ in range(3):
        y = y * (jnp.float32(1.5) - jnp.float32(0.5) * n2 * y * y)
    return y


def _splat(v):
    return jnp.full((G,), v, jnp.int32)


UC = 128
NUC = BPW // UC


def _sc_body(u_hbm, u8_hbm, fi_hbm, users_hbm, feats_hbm, ftabf_hbm, fm_hbm,
             idxu_v, urows_v, out_v, u8_c, ubuf_v, fids_c, fid_c,
             rows_v, rows2_v, sem, sem2):
    cc = lax.axis_index("c")
    s = lax.axis_index("s")
    wid = s * NC + cc
    base = wid * BPW
    pltpu.sync_copy(u_hbm.at[pl.ds(base, BPW)], idxu_v)
    for uc in range(NUC):
        pltpu.sync_copy(u8_hbm.at[pl.ds(base + uc * UC, UC)], u8_c[uc])
    for c in range(NCHUNK):
        pltpu.sync_copy(fi_hbm.at[pl.ds((base + c * C) * F, C * F)],
                        fids_c[c])
    cps = [pltpu.async_copy(ftabf_hbm.at[fids_c[c]], fid_c[c], sem)
           for c in range(NCHUNK)]

    iot = lax.iota(jnp.int32, G)

    for uc in range(NUC):
        pltpu.async_copy(users_hbm.at[u8_c[uc]], ubuf_v, sem2).wait()

        def uext(g, carry):
            lrows = uc * UC + g * G + iot
            uval = plsc.load_gather(idxu_v, [lrows])
            sub = (uval & jnp.int32(7)) * jnp.int32(D)
            brows = g * G + iot
            for d in range(D):
                v = plsc.load_gather(ubuf_v, [brows, sub + d])
                plsc.store_scatter(urows_v, [lrows, _splat(d)], v)
            return carry

        lax.fori_loop(0, UC // G, uext, 0)

    for cp in cps:
        cp.wait()

    bufs = (rows_v, rows2_v)
    cp = pltpu.async_copy(feats_hbm.at[fid_c[0]], bufs[0], sem2)
    for c in range(NCHUNK):
        nxt = None
        if c + 1 < NCHUNK:
            nxt = pltpu.async_copy(feats_hbm.at[fid_c[c + 1]],
                                   bufs[(c + 1) % 2], sem2)
        cp.wait()
        cp = nxt
        rbuf = bufs[c % 2]

        def group(g, carry):
            lane_rows = c * C + g * G + iot
            row0 = g * (G * F) + F * iot

            def feat(f, sq):
                sd, qd = sq
                rows = row0 + f
                e = [plsc.load_gather(rbuf, [rows, _splat(d)])
                     for d in range(D)]
                n2 = e[0] * e[0]
                for d in range(1, D):
                    n2 = n2 + e[d] * e[d]
                scale = jnp.minimum(jnp.float32(1.0), _rsqrt16(n2))
                t = [scale * e[d] for d in range(D)]
                return (tuple(sd[d] + t[d] for d in range(D)),
                        tuple(qd[d] + t[d] * t[d] for d in range(D)))

            zeros = tuple(jnp.zeros((G,), jnp.float32) for _ in range(D))
            sd, qd = lax.fori_loop(0, F, feat, (zeros, zeros))

            e = [plsc.load_gather(urows_v, [lane_rows, _splat(d)])
                 for d in range(D)]
            n2 = e[0] * e[0]
            for d in range(1, D):
                n2 = n2 + e[d] * e[d]
            scale = jnp.minimum(jnp.float32(1.0), _rsqrt16(n2))
            for d in range(D):
                t = scale * e[d]
                fmv = (sd[d] + t) * (sd[d] + t) - (qd[d] + t * t)
                plsc.store_scatter(out_v, [lane_rows, _splat(d)], fmv)
            return carry

        lax.fori_loop(0, CG, group, 0)

    pltpu.sync_copy(out_v, fm_hbm.at[pl.ds(base, BPW)])


@functools.partial(
    pl.kernel,
    out_type=jax.ShapeDtypeStruct((B, D), jnp.float32),
    mesh=plsc.VectorSubcoreMesh(core_axis_name="c", subcore_axis_name="s",
                                num_cores=NC, num_subcores=NS),
    scratch_types=[
        pltpu.VMEM((BPW,), jnp.int32),
        pltpu.VMEM((BPW, D), jnp.float32),
        pltpu.VMEM((BPW, D), jnp.float32),
        [pltpu.VMEM((UC,), jnp.int32) for _ in range(BPW // UC)],
        pltpu.VMEM((UC, 8 * D), jnp.float32),
        [pltpu.VMEM((C * F,), jnp.int32) for _ in range(NCHUNK)],
        [pltpu.VMEM((C * F,), jnp.int32) for _ in range(NCHUNK)],
        pltpu.VMEM((C * F, D), jnp.float32),
        pltpu.VMEM((C * F, D), jnp.float32),
        pltpu.SemaphoreType.DMA,
        pltpu.SemaphoreType.DMA,
    ],
    compiler_params=pltpu.CompilerParams(use_tc_tiling_on_sc=False,
                                         needs_layout_passes=False),
)
def _sc_gather_fm(u_hbm, u8_hbm, fi_hbm, users_hbm, feats_hbm, ftabf_hbm,
                  fm_hbm, *rest):
    _sc_body(u_hbm, u8_hbm, fi_hbm, users_hbm, feats_hbm, ftabf_hbm, fm_hbm,
             *rest)


def _mlp_body(fm_ref, w1_ref, b1_ref, w2_ref, b2_ref, w3_ref, b3_ref, o_ref):
    h = jnp.maximum(
        jnp.dot(fm_ref[...], w1_ref[...],
                preferred_element_type=jnp.float32) + b1_ref[...], 0.0)
    h = jnp.maximum(
        jnp.dot(h, w2_ref[...],
                preferred_element_type=jnp.float32) + b2_ref[...], 0.0)
    z = jnp.dot(h, w3_ref[...], preferred_element_type=jnp.float32) \
        + b3_ref[...]
    o_ref[...] = 1.0 / (1.0 + jnp.exp(-z))


def kernel(u, i, users, item_features, item_feat_table,
           W1, b1, W2, b2, W3, b3):
    flat_ids = (i[:, None] * F
                + jnp.arange(F, dtype=i.dtype)[None, :]).reshape(B * F)
    fm = _sc_gather_fm(u, u // 8, flat_ids,
                       users.reshape(N_USERS // 8, 8 * D), item_features,
                       item_feat_table.reshape(-1))
    out = pl.pallas_call(
        _mlp_body,
        out_shape=jax.ShapeDtypeStruct((B, 1), jnp.float32),
    )(fm, W1, b1, W2, b2, W3, b3)
    return jnp.squeeze(out, axis=-1)


# speedup vs baseline: 1.0027x; 1.0027x over previous
import functools

import jax
import jax.numpy as jnp
from jax import lax
from jax.experimental import pallas as pl
from jax.experimental.pallas import tpu as pltpu
from jax.experimental.pallas import tpu_sc as plsc

B = 16384
D = 16
F = 26
NC = 2
NS = 16
NW = NC * NS
BPW = B // NW
G = 16
C = 64
NCHUNK = BPW // C
CG = C // G


def _rsqrt16(n2):
    i = plsc.bitcast(n2, jnp.int32)
    y = plsc.bitcast(jnp.int32(0x5F3759DF) - jnp.right_shift(i, 1),
                     jnp.float32)
    for _ in range(3):
        y = y * (jnp.float32(1.5) - jnp.float32(0.5) * n2 * y * y)
    return y


def _splat(v):
    return jnp.full((G,), v, jnp.int32)


def _sc_body(u_hbm, fi_hbm, users_hbm, feats_hbm, ftabf_hbm, fm_hbm,
             idxu_v, urows_v, out_v, fids_c, fid_c, rows_v, rows2_v,
             sem, sem2):
    cc = lax.axis_index("c")
    s = lax.axis_index("s")
    wid = s * NC + cc
    base = wid * BPW
    pltpu.sync_copy(u_hbm.at[pl.ds(base, BPW)], idxu_v)
    for c in range(NCHUNK):
        pltpu.sync_copy(fi_hbm.at[pl.ds((base + c * C) * F, C * F)],
                        fids_c[c])
    cp_u = pltpu.async_copy(users_hbm.at[idxu_v], urows_v, sem)
    cps = [pltpu.async_copy(ftabf_hbm.at[fids_c[c]], fid_c[c], sem)
           for c in range(NCHUNK)]
    cp_u.wait()
    for cp in cps:
        cp.wait()

    iot = lax.iota(jnp.int32, G)

    bufs = (rows_v, rows2_v)
    cp = pltpu.async_copy(feats_hbm.at[fid_c[0]], bufs[0], sem2)
    for c in range(NCHUNK):
        nxt = None
        if c + 1 < NCHUNK:
            nxt = pltpu.async_copy(feats_hbm.at[fid_c[c + 1]],
                                   bufs[(c + 1) % 2], sem2)
        cp.wait()
        cp = nxt
        rbuf = bufs[c % 2]

        def group(g, carry):
            lane_rows = c * C + g * G + iot
            row0 = g * (G * F) + F * iot

            def feat(f, sq):
                sd, qd = sq
                rows = row0 + f
                e = [plsc.load_gather(rbuf, [rows, _splat(d)])
                     for d in range(D)]
                n2 = e[0] * e[0]
                for d in range(1, D):
                    n2 = n2 + e[d] * e[d]
                scale = jnp.minimum(jnp.float32(1.0), _rsqrt16(n2))
                t = [scale * e[d] for d in range(D)]
                return (tuple(sd[d] + t[d] for d in range(D)),
                        tuple(qd[d] + t[d] * t[d] for d in range(D)))

            zeros = tuple(jnp.zeros((G,), jnp.float32) for _ in range(D))
            sd, qd = lax.fori_loop(0, F, feat, (zeros, zeros))

            e = [plsc.load_gather(urows_v, [lane_rows, _splat(d)])
                 for d in range(D)]
            n2 = e[0] * e[0]
            for d in range(1, D):
                n2 = n2 + e[d] * e[d]
            scale = jnp.minimum(jnp.float32(1.0), _rsqrt16(n2))
            for d in range(D):
                t = scale * e[d]
                fmv = (sd[d] + t) * (sd[d] + t) - (qd[d] + t * t)
                plsc.store_scatter(out_v, [lane_rows, _splat(d)], fmv)
            return carry

        lax.fori_loop(0, CG, group, 0)

    pltpu.sync_copy(out_v, fm_hbm.at[pl.ds(base, BPW)])


@functools.partial(
    pl.kernel,
    out_type=jax.ShapeDtypeStruct((B, D), jnp.float32),
    mesh=plsc.VectorSubcoreMesh(core_axis_name="c", subcore_axis_name="s",
                                num_cores=NC, num_subcores=NS),
    scratch_types=[
        pltpu.VMEM((BPW,), jnp.int32),
        pltpu.VMEM((BPW, D), jnp.float32),
        pltpu.VMEM((BPW, D), jnp.float32),
        [pltpu.VMEM((C * F,), jnp.int32) for _ in range(NCHUNK)],
        [pltpu.VMEM((C * F,), jnp.int32) for _ in range(NCHUNK)],
        pltpu.VMEM((C * F, D), jnp.float32),
        pltpu.VMEM((C * F, D), jnp.float32),
        pltpu.SemaphoreType.DMA,
        pltpu.SemaphoreType.DMA,
    ],
    compiler_params=pltpu.CompilerParams(use_tc_tiling_on_sc=False,
                                         needs_layout_passes=False),
)
def _sc_gather_fm(u_hbm, fi_hbm, users_hbm, feats_hbm, ftabf_hbm, fm_hbm,
                  *rest):
    _sc_body(u_hbm, fi_hbm, users_hbm, feats_hbm, ftabf_hbm, fm_hbm, *rest)


def _mlp_body(fm_ref, w1_ref, b1_ref, w2_ref, b2_ref, w3_ref, b3_ref, o_ref):
    h = jnp.maximum(
        jnp.dot(fm_ref[...], w1_ref[...],
                preferred_element_type=jnp.float32) + b1_ref[...], 0.0)
    h = jnp.maximum(
        jnp.dot(h, w2_ref[...],
                preferred_element_type=jnp.float32) + b2_ref[...], 0.0)
    z = jnp.dot(h, w3_ref[...], preferred_element_type=jnp.float32) \
        + b3_ref[...]
    o_ref[...] = 1.0 / (1.0 + jnp.exp(-z))


def kernel(u, i, users, item_features, item_feat_table,
           W1, b1, W2, b2, W3, b3):
    flat_ids = (i[:, None] * F
                + jnp.arange(F, dtype=i.dtype)[None, :]).reshape(B * F)
    fm = _sc_gather_fm(u, flat_ids, users, item_features,
                       item_feat_table.reshape(-1))
    out = pl.pallas_call(
        _mlp_body,
        out_shape=jax.ShapeDtypeStruct((B, 1), jnp.float32),
    )(fm, W1, b1, W2, b2, W3, b3)
    return jnp.squeeze(out, axis=-1)
